# baseline (speedup 1.0000x reference)
import functools

import jax
import jax.numpy as jnp
from jax import lax
from jax.experimental import pallas as pl
from jax.experimental.pallas import tpu as pltpu
from jax.experimental.pallas import tpu_sc as plsc

NC = 2
NS = 16
NW = NC * NS

GRP = 128
K = 4
CHUNK = K * GRP


def _embed_body(idx_hbm, table_hbm, out_hbm, idx_v, rows_v, sem):
    n_grp_rows, _ = idx_hbm.shape
    d = table_hbm.shape[1]
    wid = lax.axis_index("s") * NC + lax.axis_index("c")
    groups_per_w = n_grp_rows // NW
    iters = groups_per_w // K

    def step(g, carry):
        gbase = wid * groups_per_w + g * K
        rbase = gbase * GRP
        pltpu.sync_copy(idx_hbm.at[pl.ds(gbase, K)], idx_v)
        copies = []
        for j in range(K):
            copies.append(
                pltpu.async_copy(
                    table_hbm.at[idx_v.at[j]],
                    rows_v.at[pl.ds(j * GRP, GRP)],
                    sem,
                )
            )
        for c in copies:
            c.wait()
        pltpu.sync_copy(rows_v, out_hbm.at[pl.ds(rbase, CHUNK)])
        return carry

    lax.fori_loop(0, iters, step, 0)


def _embed_call(idx_2d, weight, total_rows):
    d = weight.shape[1]
    mesh = plsc.VectorSubcoreMesh(
        core_axis_name="c", subcore_axis_name="s", num_cores=NC, num_subcores=NS
    )
    return pl.kernel(
        _embed_body,
        out_type=jax.ShapeDtypeStruct((total_rows, d), jnp.float32),
        mesh=mesh,
        scratch_types=[
            pltpu.VMEM((K, GRP), jnp.int32),
            pltpu.VMEM((CHUNK, d), jnp.float32),
            pltpu.SemaphoreType.DMA,
        ],
        compiler_params=pltpu.CompilerParams(use_tc_tiling_on_sc=False),
    )(idx_2d, weight)


def kernel(token_ids, weight):
    b, l = token_ids.shape
    total = b * l
    idx_2d = token_ids.reshape(total // GRP, GRP).astype(jnp.int32)
    out = _embed_call(idx_2d, weight, total)
    return out.reshape(b, l, weight.shape[1])


# speedup vs baseline: 1.0361x; 1.0361x over previous
import functools

import jax
import jax.numpy as jnp
from jax import lax
from jax.experimental import pallas as pl
from jax.experimental.pallas import tpu as pltpu
from jax.experimental.pallas import tpu_sc as plsc

NC = 2
NS = 16
NW = NC * NS

GRP = 128
K = 4
CHUNK = K * GRP


def _embed_body(idx_hbm, table_hbm, out_hbm, idx_v, rows_v, sem0, sem1):
    n_grp_rows, _ = idx_hbm.shape
    d = table_hbm.shape[1]
    wid = lax.axis_index("s") * NC + lax.axis_index("c")
    groups_per_w = n_grp_rows // NW
    iters = groups_per_w // K
    gbase0 = wid * groups_per_w
    sems = (sem0, sem1)

    def fire(g, b):
        pltpu.sync_copy(idx_hbm.at[pl.ds(gbase0 + g * K, K)], idx_v.at[b])
        for j in range(K):
            pltpu.async_copy(
                table_hbm.at[idx_v.at[b].at[j]],
                rows_v.at[b].at[pl.ds(j * GRP, GRP)],
                sems[b],
            )

    def drain(b):
        for j in range(K):
            pltpu.make_async_copy(
                table_hbm.at[idx_v.at[b].at[j]],
                rows_v.at[b].at[pl.ds(j * GRP, GRP)],
                sems[b],
            ).wait()

    fire(0, 0)

    def step(g2, carry):
        for b in range(2):
            g = g2 * 2 + b
            nxt = 1 - b

            @pl.when(g + 1 < iters)
            def _():
                fire(g + 1, nxt)

            drain(b)
            pltpu.sync_copy(
                rows_v.at[b],
                out_hbm.at[pl.ds((gbase0 + g * K) * GRP, CHUNK)],
            )
        return carry

    lax.fori_loop(0, iters // 2, step, 0)


def _embed_call(idx_2d, weight, total_rows):
    d = weight.shape[1]
    mesh = plsc.VectorSubcoreMesh(
        core_axis_name="c", subcore_axis_name="s", num_cores=NC, num_subcores=NS
    )
    return pl.kernel(
        _embed_body,
        out_type=jax.ShapeDtypeStruct((total_rows, d), jnp.float32),
        mesh=mesh,
        scratch_types=[
            pltpu.VMEM((2, K, GRP), jnp.int32),
            pltpu.VMEM((2, CHUNK, d), jnp.float32),
            pltpu.SemaphoreType.DMA,
            pltpu.SemaphoreType.DMA,
        ],
        compiler_params=pltpu.CompilerParams(use_tc_tiling_on_sc=False),
    )(idx_2d, weight)


def kernel(token_ids, weight):
    b, l = token_ids.shape
    total = b * l
    idx_2d = token_ids.reshape(total // GRP, GRP).astype(jnp.int32)
    out = _embed_call(idx_2d, weight, total)
    return out.reshape(b, l, weight.shape[1])
